# baseline (speedup 1.0000x reference)
import jax
import jax.numpy as jnp
from jax import lax
from jax.experimental import pallas as pl
from jax.experimental.pallas import tpu as pltpu
from jax.experimental.pallas import tpu_sc as plsc

HIDDEN = 64
BATCH = 16384
SEQ = 50
NC, NS, LANES = 2, 16, 16
NW = NC * NS
ROWS_PER_W = BATCH // NW
CHUNK = 128
NCHUNKS = ROWS_PER_W // CHUNK
NVH = HIDDEN // LANES


def _sc_body(xt_hbm, table_hbm, out_hbm, idx_v, acc_v, acc2_v, sem):
    wid = lax.axis_index("s") * NC + lax.axis_index("c")
    wbase = wid * ROWS_PER_W
    inv = jnp.float32(1.0 / SEQ)

    def chunk_body(i, carry):
        rowbase = pl.multiple_of(wbase + i * CHUNK, CHUNK)
        pltpu.sync_copy(xt_hbm.at[:, pl.ds(rowbase, CHUNK)], idx_v)
        cp0 = pltpu.async_copy(table_hbm.at[idx_v.at[0]], acc_v, sem)
        cp1 = pltpu.async_copy(table_hbm.at[idx_v.at[1]], acc2_v, sem)
        cp0.wait()
        cp1.wait()
        cps = [
            pltpu.async_copy(
                table_hbm.at[idx_v.at[s]],
                acc_v if s % 2 == 0 else acc2_v,
                sem,
                add=True,
            )
            for s in range(2, SEQ)
        ]
        for cp in cps:
            cp.wait()

        def scale_body(c, carry2):
            for h in range(NVH):
                sl = pl.ds(h * LANES, LANES)
                acc_v[c, sl] = (acc_v[c, sl] + acc2_v[c, sl]) * inv
            return carry2

        lax.fori_loop(0, CHUNK, scale_body, 0)
        pltpu.sync_copy(acc_v, out_hbm.at[pl.ds(rowbase, CHUNK)])
        return carry

    lax.fori_loop(0, NCHUNKS, chunk_body, 0)


def kernel(x, table):
    xt = x.astype(jnp.int32).T
    k = pl.kernel(
        _sc_body,
        out_type=jax.ShapeDtypeStruct((BATCH, HIDDEN), jnp.float32),
        mesh=plsc.VectorSubcoreMesh(core_axis_name="c", subcore_axis_name="s"),
        compiler_params=pltpu.CompilerParams(use_tc_tiling_on_sc=False),
        scratch_types=[
            pltpu.VMEM((SEQ, CHUNK), jnp.int32),
            pltpu.VMEM((CHUNK, HIDDEN), jnp.float32),
            pltpu.VMEM((CHUNK, HIDDEN), jnp.float32),
            pltpu.SemaphoreType.DMA,
        ],
    )
    return k(xt, table)


# speedup vs baseline: 1.0051x; 1.0051x over previous
import jax
import jax.numpy as jnp
from jax import lax
from jax.experimental import pallas as pl
from jax.experimental.pallas import tpu as pltpu
from jax.experimental.pallas import tpu_sc as plsc

HIDDEN = 64
BATCH = 16384
SEQ = 50
NC, NS, LANES = 2, 16, 16
NW = NC * NS
ROWS_PER_W = BATCH // NW
CHUNK = 512
NCHUNKS = ROWS_PER_W // CHUNK
NVH = HIDDEN // LANES


def _sc_body(xt_hbm, table_hbm, out_hbm, idx_v, acc_v, acc2_v, sem):
    wid = lax.axis_index("s") * NC + lax.axis_index("c")
    wbase = wid * ROWS_PER_W
    inv = jnp.float32(1.0 / SEQ)

    def chunk_body(i, carry):
        rowbase = pl.multiple_of(wbase + i * CHUNK, CHUNK)
        pltpu.sync_copy(xt_hbm.at[:, pl.ds(rowbase, CHUNK)], idx_v)
        cp0 = pltpu.async_copy(table_hbm.at[idx_v.at[0]], acc_v, sem)
        cp1 = pltpu.async_copy(table_hbm.at[idx_v.at[1]], acc2_v, sem)
        cp0.wait()
        cp1.wait()
        cps = [
            pltpu.async_copy(
                table_hbm.at[idx_v.at[s]],
                acc_v if s % 2 == 0 else acc2_v,
                sem,
                add=True,
            )
            for s in range(2, SEQ)
        ]
        for cp in cps:
            cp.wait()

        def scale_body(c, carry2):
            for h in range(NVH):
                sl = pl.ds(h * LANES, LANES)
                acc_v[c, sl] = (acc_v[c, sl] + acc2_v[c, sl]) * inv
            return carry2

        lax.fori_loop(0, CHUNK, scale_body, 0)
        pltpu.sync_copy(acc_v, out_hbm.at[pl.ds(rowbase, CHUNK)])
        return carry

    lax.fori_loop(0, NCHUNKS, chunk_body, 0)


def kernel(x, table):
    xt = x.astype(jnp.int32).T
    k = pl.kernel(
        _sc_body,
        out_type=jax.ShapeDtypeStruct((BATCH, HIDDEN), jnp.float32),
        mesh=plsc.VectorSubcoreMesh(core_axis_name="c", subcore_axis_name="s"),
        compiler_params=pltpu.CompilerParams(use_tc_tiling_on_sc=False),
        scratch_types=[
            pltpu.VMEM((SEQ, CHUNK), jnp.int32),
            pltpu.VMEM((CHUNK, HIDDEN), jnp.float32),
            pltpu.VMEM((CHUNK, HIDDEN), jnp.float32),
            pltpu.SemaphoreType.DMA,
        ],
    )
    return k(xt, table)


# speedup vs baseline: 1.0110x; 1.0059x over previous
import jax
import jax.numpy as jnp
from jax import lax
from jax.experimental import pallas as pl
from jax.experimental.pallas import tpu as pltpu
from jax.experimental.pallas import tpu_sc as plsc

HIDDEN = 64
BATCH = 16384
SEQ = 50
NC, NS, LANES = 2, 16, 16
NW = NC * NS
ROWS_PER_W = BATCH // NW
NVH = HIDDEN // LANES


def _sc_body(xt_hbm, table_hbm, out_hbm, idx_v, acc_v, sem):
    wid = lax.axis_index("s") * NC + lax.axis_index("c")
    wbase = pl.multiple_of(wid * ROWS_PER_W, ROWS_PER_W)
    inv = jnp.float32(1.0 / SEQ)
    zero = jnp.zeros((LANES,), jnp.float32)

    pltpu.sync_copy(xt_hbm.at[:, pl.ds(wbase, ROWS_PER_W)], idx_v)

    def zero_body(c, carry):
        for h in range(NVH):
            acc_v[c, pl.ds(h * LANES, LANES)] = zero
        return carry

    lax.fori_loop(0, ROWS_PER_W, zero_body, 0)

    cps = [
        pltpu.async_copy(table_hbm.at[idx_v.at[s]], acc_v, sem, add=True)
        for s in range(SEQ)
    ]
    for cp in cps:
        cp.wait()

    def scale_body(c, carry):
        for h in range(NVH):
            sl = pl.ds(h * LANES, LANES)
            acc_v[c, sl] = acc_v[c, sl] * inv
        return carry

    lax.fori_loop(0, ROWS_PER_W, scale_body, 0)
    pltpu.sync_copy(acc_v, out_hbm.at[pl.ds(wbase, ROWS_PER_W)])


def kernel(x, table):
    xt = x.astype(jnp.int32).T
    k = pl.kernel(
        _sc_body,
        out_type=jax.ShapeDtypeStruct((BATCH, HIDDEN), jnp.float32),
        mesh=plsc.VectorSubcoreMesh(core_axis_name="c", subcore_axis_name="s"),
        compiler_params=pltpu.CompilerParams(use_tc_tiling_on_sc=False),
        scratch_types=[
            pltpu.VMEM((SEQ, ROWS_PER_W), jnp.int32),
            pltpu.VMEM((ROWS_PER_W, HIDDEN), jnp.float32),
            pltpu.SemaphoreType.DMA,
        ],
    )
    return k(xt, table)
